# Initial kernel scaffold; baseline (speedup 1.0000x reference)
import jax
import jax.numpy as jnp
from jax.experimental import pallas as pl


def kernel(edge_logits, alpha, beta, t, cell_idx, edge_idx):
    raise NotImplementedError("write your pallas kernel here")


# speedup vs baseline: 1.0529x; 1.0529x over previous
import jax
import jax.numpy as jnp
from jax import lax
from jax.experimental import pallas as pl
from jax.experimental.pallas import tpu as pltpu
from jax.experimental.pallas import tpu_sc as plsc

B = 16384
E = 128
NC = 2
NS = 16
NW = NC * NS
BPW = B // NW
CHUNK = 128
NCHUNK = BPW // CHUNK
LANES = 16
NGRP = BPW // LANES


def _select_elems(buf, edge_flat, dst, goff):
    for k in range(CHUNK // LANES):
        rl = lax.iota(jnp.int32, LANES) + (k * LANES)
        e = edge_flat[pl.ds(goff * LANES + k * LANES, LANES)]
        dst[pl.ds(goff * LANES + k * LANES, LANES)] = plsc.load_gather(
            buf, [rl, e])


def _sc_body(logits_hbm, alpha_hbm, beta_hbm, cell_hbm, edge_hbm,
             usel_out, s_out, a_out, b_out,
             cell_v, edge_v, edge_flat, rows_v, abuf0, abuf1,
             usel_v, s_v, a_v, b_v, semr, semab):
    wid = lax.axis_index("s") * NC + lax.axis_index("c")
    pltpu.sync_copy(cell_hbm.at[pl.ds(wid * NCHUNK, NCHUNK)], cell_v)
    pltpu.sync_copy(edge_hbm.at[pl.ds(wid * NCHUNK, NCHUNK)], edge_v)
    for g in range(NGRP):
        edge_flat[pl.ds(g * LANES, LANES)] = edge_v[
            g // 8, pl.ds((g % 8) * LANES, LANES)]
    row_copies = []
    for j in range(NCHUNK):
        row_copies.append(pltpu.async_copy(
            logits_hbm.at[cell_v.at[j]], rows_v.at[pl.ds(j * CHUNK, CHUNK)],
            semr))
    for tbl, dst in ((alpha_hbm, a_v), (beta_hbm, b_v)):
        bufs = (abuf0, abuf1)
        pend = [pltpu.async_copy(tbl.at[cell_v.at[0]], bufs[0], semab),
                pltpu.async_copy(tbl.at[cell_v.at[1]], bufs[1], semab)]
        for j in range(NCHUNK):
            pend[j % 2].wait()
            if j + 2 < NCHUNK:
                pass
            _select_elems(bufs[j % 2], edge_flat, dst, j * (CHUNK // LANES))
            if j + 2 < NCHUNK:
                pend[j % 2] = pltpu.async_copy(
                    tbl.at[cell_v.at[j + 2]], bufs[j % 2], semab)
    for c in row_copies:
        c.wait()
    def _grp(g, carry):
        rl = lax.iota(jnp.int32, LANES) + g * LANES
        acc = [plsc.load_gather(rows_v, [rl, jnp.full((LANES,), c,
                                                      jnp.int32)])
               for c in range(4)]
        for c in range(4, E, 4):
            for q in range(4):
                v = plsc.load_gather(rows_v, [rl, jnp.full((LANES,), c + q,
                                                           jnp.int32)])
                acc[q] = jnp.maximum(acc[q], v)
        m = jnp.maximum(jnp.maximum(acc[0], acc[1]),
                        jnp.maximum(acc[2], acc[3]))
        sacc = [jnp.exp(plsc.load_gather(rows_v,
                                         [rl, jnp.full((LANES,), c,
                                                       jnp.int32)]) - m)
                for c in range(4)]
        for c in range(4, E, 4):
            for q in range(4):
                v = plsc.load_gather(rows_v, [rl, jnp.full((LANES,), c + q,
                                                           jnp.int32)])
                sacc[q] = sacc[q] + jnp.exp(v - m)
        s = (sacc[0] + sacc[1]) + (sacc[2] + sacc[3])
        e = edge_flat[pl.ds(g * LANES, LANES)]
        sel = plsc.load_gather(rows_v, [rl, e])
        usel_v[pl.ds(g * LANES, LANES)] = sel - m
        s_v[pl.ds(g * LANES, LANES)] = s
        return carry

    lax.fori_loop(0, NGRP, _grp, 0)
    base = wid * BPW
    pltpu.sync_copy(usel_v, usel_out.at[pl.ds(base, BPW)])
    pltpu.sync_copy(s_v, s_out.at[pl.ds(base, BPW)])
    pltpu.sync_copy(a_v, a_out.at[pl.ds(base, BPW)])
    pltpu.sync_copy(b_v, b_out.at[pl.ds(base, BPW)])


_sc_gather = pl.kernel(
    _sc_body,
    out_type=(
        jax.ShapeDtypeStruct((B,), jnp.float32),
        jax.ShapeDtypeStruct((B,), jnp.float32),
        jax.ShapeDtypeStruct((B,), jnp.float32),
        jax.ShapeDtypeStruct((B,), jnp.float32),
    ),
    mesh=plsc.VectorSubcoreMesh(core_axis_name="c", subcore_axis_name="s"),
    compiler_params=pltpu.CompilerParams(needs_layout_passes=False),
    scratch_types=[
        pltpu.VMEM((NCHUNK, CHUNK), jnp.int32),
        pltpu.VMEM((NCHUNK, CHUNK), jnp.int32),
        pltpu.VMEM((BPW,), jnp.int32),
        pltpu.VMEM((BPW, E), jnp.float32),
        pltpu.VMEM((CHUNK, E), jnp.float32),
        pltpu.VMEM((CHUNK, E), jnp.float32),
        pltpu.VMEM((BPW,), jnp.float32),
        pltpu.VMEM((BPW,), jnp.float32),
        pltpu.VMEM((BPW,), jnp.float32),
        pltpu.VMEM((BPW,), jnp.float32),
        pltpu.SemaphoreType.DMA,
        pltpu.SemaphoreType.DMA,
    ],
)


_HALF_LOG_2PI = 0.9189385332046727


def _lgamma(x):
    x4 = x + 4.0
    z = 1.0 / x4
    z2 = z * z
    series = z * (0.08333333333333333 +
                  z2 * (-0.002777777777777778 + z2 * 0.0007936507936507937))
    st = (x4 - 0.5) * jnp.log(x4) - x4 + _HALF_LOG_2PI + series
    prod = x * (x + 1.0) * (x + 2.0) * (x + 3.0)
    return st - jnp.log(prod)


def _final_body(usel_ref, s_ref, a_ref, b_ref, t_ref, o_ref):
    usel = usel_ref[...]
    s = s_ref[...]
    a = a_ref[...]
    b = b_ref[...]
    t = t_ref[...]
    p = jnp.exp(usel) / s
    log_edge = jnp.log(p + 1e-10)
    log_t = ((a - 1.0) * jnp.log(t) + (b - 1.0) * jnp.log1p(-t)
             + _lgamma(a + b) - _lgamma(a) - _lgamma(b))
    o_ref[...] = log_edge + log_t


def _final_call(usel2, s2, a2, b2, t2):
    return pl.pallas_call(
        _final_body,
        out_shape=jax.ShapeDtypeStruct((B // E, E), jnp.float32),
    )(usel2, s2, a2, b2, t2)


def kernel(edge_logits, alpha, beta, t, cell_idx, edge_idx):
    cell = cell_idx.astype(jnp.int32).reshape(B // CHUNK, CHUNK)
    edge = edge_idx.astype(jnp.int32).reshape(B // CHUNK, CHUNK)
    usel, s, a_g, b_g = _sc_gather(edge_logits, alpha, beta, cell, edge)
    out2 = _final_call(usel.reshape(B // E, E), s.reshape(B // E, E),
                       a_g.reshape(B // E, E), b_g.reshape(B // E, E),
                       t.astype(jnp.float32).reshape(B // E, E))
    return out2.reshape(B)


# speedup vs baseline: 1.5233x; 1.4468x over previous
import jax
import jax.numpy as jnp
from jax import lax
from jax.experimental import pallas as pl
from jax.experimental.pallas import tpu as pltpu
from jax.experimental.pallas import tpu_sc as plsc

B = 16384
E = 128
NC = 2
NS = 16
NW = NC * NS
BPW = B // NW
CHUNK = 128
NCHUNK = BPW // CHUNK
LANES = 16
RBLK = 1024
NSTEP = B // RBLK


def _select_elems(buf, edge_v, j, dst):
    for k in range(CHUNK // LANES):
        rl = lax.iota(jnp.int32, LANES) + (k * LANES)
        e = edge_v[j, pl.ds(k * LANES, LANES)]
        dst[pl.ds(j * CHUNK + k * LANES, LANES)] = plsc.load_gather(
            buf, [rl, e])


def _sc_body(logits_hbm, alpha_hbm, beta_hbm, cell_hbm, edge_hbm,
             rows_out, sel_out, a_out, b_out,
             cell_v, edge_v, rows_v, abuf0, abuf1,
             sel_v, a_v, b_v, semr, semab):
    wid = lax.axis_index("s") * NC + lax.axis_index("c")
    pltpu.sync_copy(cell_hbm.at[pl.ds(wid * NCHUNK, NCHUNK)], cell_v)
    pltpu.sync_copy(edge_hbm.at[pl.ds(wid * NCHUNK, NCHUNK)], edge_v)
    row_copies = []
    for j in range(NCHUNK):
        row_copies.append(pltpu.async_copy(
            logits_hbm.at[cell_v.at[j]], rows_v.at[pl.ds(j * CHUNK, CHUNK)],
            semr))
    for tbl, dst in ((alpha_hbm, a_v), (beta_hbm, b_v)):
        bufs = (abuf0, abuf1)
        pend = [pltpu.async_copy(tbl.at[cell_v.at[0]], bufs[0], semab),
                pltpu.async_copy(tbl.at[cell_v.at[1]], bufs[1], semab)]
        for j in range(NCHUNK):
            pend[j % 2].wait()
            _select_elems(bufs[j % 2], edge_v, j, dst)
            if j + 2 < NCHUNK:
                pend[j % 2] = pltpu.async_copy(
                    tbl.at[cell_v.at[j + 2]], bufs[j % 2], semab)
    for c in row_copies:
        c.wait()
    for j in range(NCHUNK):
        _select_elems(rows_v.at[pl.ds(j * CHUNK, CHUNK)], edge_v, j, sel_v)
    base = wid * BPW
    pltpu.sync_copy(rows_v, rows_out.at[pl.ds(base, BPW)])
    pltpu.sync_copy(sel_v, sel_out.at[pl.ds(base, BPW)])
    pltpu.sync_copy(a_v, a_out.at[pl.ds(base, BPW)])
    pltpu.sync_copy(b_v, b_out.at[pl.ds(base, BPW)])


_sc_gather = pl.kernel(
    _sc_body,
    out_type=(
        jax.ShapeDtypeStruct((B, E), jnp.float32),
        jax.ShapeDtypeStruct((B,), jnp.float32),
        jax.ShapeDtypeStruct((B,), jnp.float32),
        jax.ShapeDtypeStruct((B,), jnp.float32),
    ),
    mesh=plsc.VectorSubcoreMesh(core_axis_name="c", subcore_axis_name="s"),
    compiler_params=pltpu.CompilerParams(needs_layout_passes=False),
    scratch_types=[
        pltpu.VMEM((NCHUNK, CHUNK), jnp.int32),
        pltpu.VMEM((NCHUNK, CHUNK), jnp.int32),
        pltpu.VMEM((BPW, E), jnp.float32),
        pltpu.VMEM((CHUNK, E), jnp.float32),
        pltpu.VMEM((CHUNK, E), jnp.float32),
        pltpu.VMEM((BPW,), jnp.float32),
        pltpu.VMEM((BPW,), jnp.float32),
        pltpu.VMEM((BPW,), jnp.float32),
        pltpu.SemaphoreType.DMA,
        pltpu.SemaphoreType.DMA,
    ],
)


_HALF_LOG_2PI = 0.9189385332046727


def _lgamma(x):
    x4 = x + 4.0
    z = 1.0 / x4
    z2 = z * z
    series = z * (0.08333333333333333 +
                  z2 * (-0.002777777777777778 + z2 * 0.0007936507936507937))
    st = (x4 - 0.5) * jnp.log(x4) - x4 + _HALF_LOG_2PI + series
    prod = x * (x + 1.0) * (x + 2.0) * (x + 3.0)
    return st - jnp.log(prod)


def _tc_body(rows_ref, sel_ref, a_ref, b_ref, t_ref, o_ref, lse_s):
    g = pl.program_id(0)

    @pl.when(g < NSTEP)
    def _reduce():
        x3 = rows_ref[...].reshape(RBLK // E, E, E)
        m3 = jnp.max(x3, axis=2)
        s3 = jnp.sum(jnp.exp(x3 - m3[:, :, None]), axis=2)
        lse_s[pl.ds(g * (RBLK // E), RBLK // E), :] = m3 + jnp.log(s3)

    @pl.when(g == NSTEP)
    def _finish():
        lse = lse_s[...]
        sel = sel_ref[...]
        a = a_ref[...]
        b = b_ref[...]
        t = t_ref[...]
        p = jnp.exp(sel - lse)
        log_edge = jnp.log(p + 1e-10)
        log_t = ((a - 1.0) * jnp.log(t) + (b - 1.0) * jnp.log1p(-t)
                 + _lgamma(a + b) - _lgamma(a) - _lgamma(b))
        o_ref[...] = log_edge + log_t


def _tc_call(rows, sel2, a2, b2, t2):
    vec_spec = pl.BlockSpec((B // E, E), lambda g: (0, 0))
    return pl.pallas_call(
        _tc_body,
        grid=(NSTEP + 1,),
        in_specs=[
            pl.BlockSpec((RBLK, E), lambda g: (jnp.minimum(g, NSTEP - 1), 0)),
            vec_spec, vec_spec, vec_spec, vec_spec,
        ],
        out_specs=pl.BlockSpec((B // E, E), lambda g: (0, 0)),
        out_shape=jax.ShapeDtypeStruct((B // E, E), jnp.float32),
        scratch_shapes=[pltpu.VMEM((B // E, E), jnp.float32)],
    )(rows, sel2, a2, b2, t2)


def kernel(edge_logits, alpha, beta, t, cell_idx, edge_idx):
    cell = cell_idx.astype(jnp.int32).reshape(B // CHUNK, CHUNK)
    edge = edge_idx.astype(jnp.int32).reshape(B // CHUNK, CHUNK)
    rows, sel, a_g, b_g = _sc_gather(edge_logits, alpha, beta, cell, edge)
    out2 = _tc_call(rows, sel.reshape(B // E, E), a_g.reshape(B // E, E),
                    b_g.reshape(B // E, E),
                    t.astype(jnp.float32).reshape(B // E, E))
    return out2.reshape(B)


# speedup vs baseline: 4.4432x; 2.9168x over previous
import jax
import jax.numpy as jnp
from jax import lax
from jax.experimental import pallas as pl
from jax.experimental.pallas import tpu as pltpu
from jax.experimental.pallas import tpu_sc as plsc

B = 16384
E = 128
NC = 2
NS = 16
NW = NC * NS
BPW = B // NW
CHUNK = 128
NCHUNK = BPW // CHUNK
LANES = 16
RBLK = 4096
NSTEP = B // RBLK


def _select_elems(buf, edge_v, j, dst):
    for k in range(CHUNK // LANES):
        rl = lax.iota(jnp.int32, LANES) + (k * LANES)
        e = edge_v[j, pl.ds(k * LANES, LANES)]
        dst[pl.ds(j * CHUNK + k * LANES, LANES)] = plsc.load_gather(
            buf, [rl, e])


def _sc_body(logits_hbm, alpha_hbm, beta_hbm, cell_hbm, edge_hbm,
             rows_out, sel_out, a_out, b_out,
             cell_v, edge_v, rows_v, abuf0, abuf1,
             sel_v, a_v, b_v, semr, semab):
    wid = lax.axis_index("s") * NC + lax.axis_index("c")
    pltpu.sync_copy(cell_hbm.at[pl.ds(wid * NCHUNK, NCHUNK)], cell_v)
    pltpu.sync_copy(edge_hbm.at[pl.ds(wid * NCHUNK, NCHUNK)], edge_v)
    row_copies = []
    for j in range(NCHUNK):
        row_copies.append(pltpu.async_copy(
            logits_hbm.at[cell_v.at[j]], rows_v.at[pl.ds(j * CHUNK, CHUNK)],
            semr))
    for tbl, dst in ((alpha_hbm, a_v), (beta_hbm, b_v)):
        bufs = (abuf0, abuf1)
        pend = [pltpu.async_copy(tbl.at[cell_v.at[0]], bufs[0], semab),
                pltpu.async_copy(tbl.at[cell_v.at[1]], bufs[1], semab)]
        for j in range(NCHUNK):
            pend[j % 2].wait()
            _select_elems(bufs[j % 2], edge_v, j, dst)
            if j + 2 < NCHUNK:
                pend[j % 2] = pltpu.async_copy(
                    tbl.at[cell_v.at[j + 2]], bufs[j % 2], semab)
    for c in row_copies:
        c.wait()
    for j in range(NCHUNK):
        _select_elems(rows_v.at[pl.ds(j * CHUNK, CHUNK)], edge_v, j, sel_v)
    base = wid * BPW
    pltpu.sync_copy(rows_v, rows_out.at[pl.ds(base, BPW)])
    pltpu.sync_copy(sel_v, sel_out.at[pl.ds(base, BPW)])
    pltpu.sync_copy(a_v, a_out.at[pl.ds(base, BPW)])
    pltpu.sync_copy(b_v, b_out.at[pl.ds(base, BPW)])


_sc_gather = pl.kernel(
    _sc_body,
    out_type=(
        jax.ShapeDtypeStruct((B, E), jnp.float32),
        jax.ShapeDtypeStruct((B,), jnp.float32),
        jax.ShapeDtypeStruct((B,), jnp.float32),
        jax.ShapeDtypeStruct((B,), jnp.float32),
    ),
    mesh=plsc.VectorSubcoreMesh(core_axis_name="c", subcore_axis_name="s"),
    compiler_params=pltpu.CompilerParams(needs_layout_passes=False),
    scratch_types=[
        pltpu.VMEM((NCHUNK, CHUNK), jnp.int32),
        pltpu.VMEM((NCHUNK, CHUNK), jnp.int32),
        pltpu.VMEM((BPW, E), jnp.float32),
        pltpu.VMEM((CHUNK, E), jnp.float32),
        pltpu.VMEM((CHUNK, E), jnp.float32),
        pltpu.VMEM((BPW,), jnp.float32),
        pltpu.VMEM((BPW,), jnp.float32),
        pltpu.VMEM((BPW,), jnp.float32),
        pltpu.SemaphoreType.DMA,
        pltpu.SemaphoreType.DMA,
    ],
)


_HALF_LOG_2PI = 0.9189385332046727


def _lgamma(x):
    x4 = x + 4.0
    z = 1.0 / x4
    z2 = z * z
    series = z * (0.08333333333333333 +
                  z2 * (-0.002777777777777778 + z2 * 0.0007936507936507937))
    st = (x4 - 0.5) * jnp.log(x4) - x4 + _HALF_LOG_2PI + series
    prod = x * (x + 1.0) * (x + 2.0) * (x + 3.0)
    return st - jnp.log(prod)


def _tc_body(rows_ref, sel_ref, a_ref, b_ref, t_ref, o_ref, lse_s):
    g = pl.program_id(0)

    @pl.when(g < NSTEP)
    def _reduce():
        x3 = rows_ref[...].reshape(RBLK // E, E, E)
        m3 = jnp.max(x3, axis=2)
        s3 = jnp.sum(jnp.exp(x3 - m3[:, :, None]), axis=2)
        lse_s[pl.ds(g * (RBLK // E), RBLK // E), :] = m3 + jnp.log(s3)

    @pl.when(g == NSTEP)
    def _finish():
        lse = lse_s[...]
        sel = sel_ref[...]
        a = a_ref[...]
        b = b_ref[...]
        t = t_ref[...]
        p = jnp.exp(sel - lse)
        log_edge = jnp.log(p + 1e-10)
        log_t = ((a - 1.0) * jnp.log(t) + (b - 1.0) * jnp.log1p(-t)
                 + _lgamma(a + b) - _lgamma(a) - _lgamma(b))
        o_ref[...] = log_edge + log_t


def _tc_call(rows, sel2, a2, b2, t2):
    vec_spec = pl.BlockSpec((B // E, E), lambda g: (0, 0))
    return pl.pallas_call(
        _tc_body,
        grid=(NSTEP + 1,),
        in_specs=[
            pl.BlockSpec((RBLK, E), lambda g: (jnp.minimum(g, NSTEP - 1), 0)),
            vec_spec, vec_spec, vec_spec, vec_spec,
        ],
        out_specs=pl.BlockSpec((B // E, E), lambda g: (0, 0)),
        out_shape=jax.ShapeDtypeStruct((B // E, E), jnp.float32),
        scratch_shapes=[pltpu.VMEM((B // E, E), jnp.float32)],
    )(rows, sel2, a2, b2, t2)


def kernel(edge_logits, alpha, beta, t, cell_idx, edge_idx):
    cell = cell_idx.astype(jnp.int32).reshape(B // CHUNK, CHUNK)
    edge = edge_idx.astype(jnp.int32).reshape(B // CHUNK, CHUNK)
    rows, sel, a_g, b_g = _sc_gather(edge_logits, alpha, beta, cell, edge)
    out2 = _tc_call(rows, sel.reshape(B // E, E), a_g.reshape(B // E, E),
                    b_g.reshape(B // E, E),
                    t.astype(jnp.float32).reshape(B // E, E))
    return out2.reshape(B)
